# Initial kernel scaffold; baseline (speedup 1.0000x reference)
import jax
import jax.numpy as jnp
from jax.experimental import pallas as pl


def kernel(x, edge_index, W1, a_src1, a_dst1, b1, W2, a_src2, a_dst2, b2, bn_gamma, bn_beta, ln_gamma, ln_beta):
    raise NotImplementedError("write your pallas kernel here")


# speedup vs baseline: 28.6455x; 28.6455x over previous
import functools

import jax
import jax.numpy as jnp
from jax import lax
from jax.experimental import pallas as pl
from jax.experimental.pallas import tpu as pltpu
from jax.experimental.pallas import tpu_sc as plsc

N = 10000
E = 320000
F = 128
ACC_W = 144
NEG = 0.2
EPS = 1e-5

NUM_CORES = 2
NUM_SUBCORES = 16
E_PER_CORE = E // NUM_CORES
E_PER_SUB = E_PER_CORE // NUM_SUBCORES
CHUNK = 80
NCHUNK = E_PER_SUB // CHUNK
N_PAD = 10240
ROWS_PER_SUB = N_PAD // NUM_SUBCORES

_HIGH = lax.Precision.HIGHEST

_GATHER_DNUMS = lax.GatherDimensionNumbers(
    offset_dims=(), collapsed_slice_dims=(0,), start_index_map=(0,))


def _splat(v, lane):
    idx = jnp.full((16, 1), lane, jnp.int32)
    return lax.gather(v, idx, _GATHER_DNUMS, slice_sizes=(1,),
                      mode=lax.GatherScatterMode.PROMISE_IN_BOUNDS)


def _dot(a, b):
    return jnp.dot(a, b, preferred_element_type=jnp.float32, precision=_HIGH)


def _make_sc_agg(heads):
    chan = F // heads
    grp_head = [(16 * g) // chan for g in range(8)]
    mesh = plsc.VectorSubcoreMesh(core_axis_name="c", subcore_axis_name="s")

    @functools.partial(
        pl.kernel,
        out_type=jax.ShapeDtypeStruct((NUM_CORES, N_PAD, ACC_W), jnp.float32),
        mesh=mesh,
        compiler_params=pltpu.CompilerParams(use_tc_tiling_on_sc=False),
        scratch_types=[
            pltpu.VMEM_SHARED((N_PAD, ACC_W), jnp.float32),
            pltpu.VMEM((CHUNK,), jnp.int32),
            pltpu.VMEM((CHUNK,), jnp.int32),
            pltpu.VMEM((CHUNK, F), jnp.float32),
            pltpu.VMEM((CHUNK, 16), jnp.float32),
            pltpu.VMEM((CHUNK, 16), jnp.float32),
            pltpu.VMEM((CHUNK, ACC_W), jnp.float32),
            pltpu.SemaphoreType.DMA,
            pltpu.SemaphoreType.DMA,
            pltpu.SemaphoreType.DMA,
        ],
    )
    def sc_agg(h_hbm, sda_hbm, sdb_hbm, src_hbm, dst_hbm, zero_hbm,
               out_hbm, acc, src_v, dst_v, hsrc_v, sa_v, sb_v, msg_v,
               sem_h, sem_a, sem_b):
        cid = lax.axis_index("c")
        sid = lax.axis_index("s")
        row0 = sid * ROWS_PER_SUB
        pltpu.sync_copy(zero_hbm.at[pl.ds(row0, ROWS_PER_SUB)],
                        acc.at[pl.ds(row0, ROWS_PER_SUB)])
        plsc.subcore_barrier()

        ebase = cid * E_PER_CORE + sid * E_PER_SUB
        lanes = lax.iota(jnp.int32, 16)
        hmask = lanes < heads

        def chunk_body(i, carry):
            base = ebase + i * CHUNK
            pltpu.sync_copy(src_hbm.at[pl.ds(base, CHUNK)], src_v)
            pltpu.sync_copy(dst_hbm.at[pl.ds(base, CHUNK)], dst_v)
            ch = pltpu.async_copy(h_hbm.at[src_v], hsrc_v, sem_h)
            ca = pltpu.async_copy(sda_hbm.at[src_v], sa_v, sem_a)
            cb = pltpu.async_copy(sdb_hbm.at[dst_v], sb_v, sem_b)
            ch.wait()
            ca.wait()
            cb.wait()

            def edge_body(j, ecarry):
                e = sa_v[j, :] + sb_v[j, :]
                e = jnp.where(e > 0, e, NEG * e)
                w = jnp.exp(e)
                msg_v[j, pl.ds(F, 16)] = jnp.where(hmask, w, 0.0)
                for g in range(8):
                    wspl = _splat(w, grp_head[g])
                    msg_v[j, pl.ds(16 * g, 16)] = (
                        hsrc_v[j, pl.ds(16 * g, 16)] * wspl)
                return ecarry

            lax.fori_loop(0, CHUNK, edge_body, 0)
            pltpu.sync_copy(msg_v, acc.at[dst_v], add=True)
            return carry

        lax.fori_loop(0, NCHUNK, chunk_body, 0)

        plsc.subcore_barrier()
        pltpu.sync_copy(acc.at[pl.ds(row0, ROWS_PER_SUB)],
                        out_hbm.at[cid, pl.ds(row0, ROWS_PER_SUB)])

    return sc_agg


_SC_CACHE = {}


def _sc_agg(heads):
    if heads not in _SC_CACHE:
        _SC_CACHE[heads] = _make_sc_agg(heads)
    return _SC_CACHE[heads]


_GRID = 10
_BLK = N // _GRID


def _row_spec(w):
    return pl.BlockSpec((_BLK, w), lambda i: (i, 0))


def _full_spec(shape):
    return pl.BlockSpec(shape, lambda i: tuple(0 for _ in shape))


def _tc1_body(x_ref, w1_ref, ma_ref, mb_ref, h_ref, sda_ref, sdb_ref):
    h = _dot(x_ref[...], w1_ref[...])
    h_ref[...] = h
    sda_ref[...] = _dot(h, ma_ref[...])
    sdb_ref[...] = _dot(h, mb_ref[...])


def _tc1(x, w1, ma, mb):
    return pl.pallas_call(
        _tc1_body,
        grid=(_GRID,),
        in_specs=[_row_spec(F), _full_spec((F, F)),
                  _full_spec((F, 16)), _full_spec((F, 16))],
        out_specs=[_row_spec(F), _row_spec(16), _row_spec(16)],
        out_shape=[
            jax.ShapeDtypeStruct((N, F), jnp.float32),
            jax.ShapeDtypeStruct((N, 16), jnp.float32),
            jax.ShapeDtypeStruct((N, 16), jnp.float32),
        ],
    )(x, w1, ma, mb)


def _tc2_body(p_ref, h1_ref, sda1_ref, b1_ref, r_ref, w2_ref, ma2_ref,
              mb2_ref, h2_ref, sda2_ref, sdb2_ref):
    p = p_ref[0] + p_ref[1]
    num = p[:, 0:F]
    den8 = p[:, F:F + 8]
    es = sda1_ref[:, 0:8] + sda1_ref[:, 8:16]
    es = jnp.where(es > 0, es, NEG * es)
    wself = jnp.exp(es)
    r = r_ref[...]
    den_exp = _dot(den8 + wself, r)
    wself_exp = _dot(wself, r)
    out = (num + h1_ref[...] * wself_exp) / (den_exp + 1e-16) + b1_ref[...]
    act = jnp.where(out > 0, out, jnp.exp(out) - 1.0)
    h2 = _dot(act, w2_ref[...])
    h2_ref[...] = h2
    sda2_ref[...] = _dot(h2, ma2_ref[...])
    sdb2_ref[...] = _dot(h2, mb2_ref[...])


def _tc2(p, h1, sda1, b1, r, w2, ma2, mb2):
    return pl.pallas_call(
        _tc2_body,
        grid=(_GRID,),
        in_specs=[
            pl.BlockSpec((NUM_CORES, _BLK, ACC_W), lambda i: (0, i, 0)),
            _row_spec(F), _row_spec(16), _full_spec((1, F)),
            _full_spec((8, F)), _full_spec((F, F)),
            _full_spec((F, 16)), _full_spec((F, 16)),
        ],
        out_specs=[_row_spec(F), _row_spec(16), _row_spec(16)],
        out_shape=[
            jax.ShapeDtypeStruct((N, F), jnp.float32),
            jax.ShapeDtypeStruct((N, 16), jnp.float32),
            jax.ShapeDtypeStruct((N, 16), jnp.float32),
        ],
    )(p, h1, sda1, b1, r, w2, ma2, mb2)


def _tc3_body(p_ref, h2_ref, sda2_ref, b2_ref, bng_ref, bnb_ref, lng_ref,
              lnb_ref, y_ref):
    p = p_ref[0] + p_ref[1]
    num = p[:, 0:F]
    den = p[:, F:F + 1]
    es = sda2_ref[:, 0:1] + sda2_ref[:, 8:9]
    es = jnp.where(es > 0, es, NEG * es)
    wself = jnp.exp(es)
    out = (num + h2_ref[...] * wself) / (den + wself + 1e-16) + b2_ref[...]
    act = jnp.where(out > 0, out, jnp.exp(out) - 1.0)
    hb = act * bng_ref[...] + bnb_ref[...]
    mu = jnp.mean(hb, axis=-1, keepdims=True)
    var = jnp.mean((hb - mu) * (hb - mu), axis=-1, keepdims=True)
    y_ref[...] = (hb - mu) / jnp.sqrt(var + EPS) * lng_ref[...] + lnb_ref[...]


def _tc3(p, h2, sda2, b2, bng, bnb, lng, lnb):
    return pl.pallas_call(
        _tc3_body,
        grid=(_GRID,),
        in_specs=[
            pl.BlockSpec((NUM_CORES, _BLK, ACC_W), lambda i: (0, i, 0)),
            _row_spec(F), _row_spec(16),
            _full_spec((1, F)), _full_spec((1, F)), _full_spec((1, F)),
            _full_spec((1, F)), _full_spec((1, F)),
        ],
        out_specs=_row_spec(F),
        out_shape=jax.ShapeDtypeStruct((N, F), jnp.float32),
    )(p, h2, sda2, b2, bng, bnb, lng, lnb)


def _logit_mats(a_src, a_dst, heads):
    chan = F // heads
    eye = jnp.eye(heads, dtype=jnp.float32)
    a_s = (a_src[:, :, None] * eye[:, None, :]).reshape(F, heads)
    a_d = (a_dst[:, :, None] * eye[:, None, :]).reshape(F, heads)
    pad = jnp.zeros((F, 8 - heads), jnp.float32)
    a_s = jnp.concatenate([a_s, pad], axis=1)
    a_d = jnp.concatenate([a_d, pad], axis=1)
    ma = jnp.concatenate([a_s, a_d], axis=1)
    mb = jnp.concatenate([a_d, a_s], axis=1)
    return ma, mb


def kernel(x, edge_index, W1, a_src1, a_dst1, b1, W2, a_src2, a_dst2, b2,
           bn_gamma, bn_beta, ln_gamma, ln_beta):
    src = edge_index[0].astype(jnp.int32)
    dst = edge_index[1].astype(jnp.int32)
    ma1, mb1 = _logit_mats(a_src1, a_dst1, 8)
    ma2, mb2 = _logit_mats(a_src2, a_dst2, 1)
    r = (jnp.eye(8, dtype=jnp.float32)[:, :, None]
         * jnp.ones((1, 1, 16), jnp.float32)).reshape(8, F)
    zeros_acc = jnp.zeros((N_PAD, ACC_W), jnp.float32)

    h1, sda1, sdb1 = _tc1(x, W1, ma1, mb1)
    p1 = _sc_agg(8)(h1, sda1, sdb1, src, dst, zeros_acc)
    h2, sda2, sdb2 = _tc2(p1, h1, sda1, b1.reshape(1, F), r, W2, ma2, mb2)
    p2 = _sc_agg(1)(h2, sda2, sdb2, src, dst, zeros_acc)
    return _tc3(p2, h2, sda2, b2.reshape(1, F), bn_gamma.reshape(1, F),
                bn_beta.reshape(1, F), ln_gamma.reshape(1, F),
                ln_beta.reshape(1, F))


# speedup vs baseline: 28.7495x; 1.0036x over previous
import functools

import jax
import jax.numpy as jnp
from jax import lax
from jax.experimental import pallas as pl
from jax.experimental.pallas import tpu as pltpu
from jax.experimental.pallas import tpu_sc as plsc

N = 10000
E = 320000
F = 128
ACC_W = 144
NEG = 0.2
EPS = 1e-5

NUM_CORES = 2
NUM_SUBCORES = 16
E_PER_CORE = E // NUM_CORES
E_PER_SUB = E_PER_CORE // NUM_SUBCORES
CHUNK = 80
UNROLL = 8
NCHUNK = E_PER_SUB // CHUNK
N_PAD = 10240
ROWS_PER_SUB = N_PAD // NUM_SUBCORES

_HIGH = lax.Precision.HIGHEST

_GATHER_DNUMS = lax.GatherDimensionNumbers(
    offset_dims=(), collapsed_slice_dims=(0,), start_index_map=(0,))


def _splat(v, lane):
    idx = jnp.full((16, 1), lane, jnp.int32)
    return lax.gather(v, idx, _GATHER_DNUMS, slice_sizes=(1,),
                      mode=lax.GatherScatterMode.PROMISE_IN_BOUNDS)


def _dot(a, b):
    return jnp.dot(a, b, preferred_element_type=jnp.float32, precision=_HIGH)


def _make_sc_agg(heads):
    chan = F // heads
    grp_head = [(16 * g) // chan for g in range(8)]
    mesh = plsc.VectorSubcoreMesh(core_axis_name="c", subcore_axis_name="s")

    @functools.partial(
        pl.kernel,
        out_type=jax.ShapeDtypeStruct((NUM_CORES, N_PAD, ACC_W), jnp.float32),
        mesh=mesh,
        compiler_params=pltpu.CompilerParams(use_tc_tiling_on_sc=False),
        scratch_types=[
            pltpu.VMEM_SHARED((N_PAD, ACC_W), jnp.float32),
            pltpu.VMEM((CHUNK,), jnp.int32),
            pltpu.VMEM((CHUNK,), jnp.int32),
            pltpu.VMEM((CHUNK, F), jnp.float32),
            pltpu.VMEM((CHUNK, 16), jnp.float32),
            pltpu.VMEM((CHUNK, 16), jnp.float32),
            pltpu.VMEM((CHUNK, ACC_W), jnp.float32),
            pltpu.SemaphoreType.DMA,
            pltpu.SemaphoreType.DMA,
            pltpu.SemaphoreType.DMA,
        ],
    )
    def sc_agg(h_hbm, sda_hbm, sdb_hbm, src_hbm, dst_hbm, zero_hbm,
               out_hbm, acc, src_v, dst_v, hsrc_v, sa_v, sb_v, msg_v,
               sem_h, sem_a, sem_b):
        cid = lax.axis_index("c")
        sid = lax.axis_index("s")
        row0 = sid * ROWS_PER_SUB
        pltpu.sync_copy(zero_hbm.at[pl.ds(row0, ROWS_PER_SUB)],
                        acc.at[pl.ds(row0, ROWS_PER_SUB)])
        plsc.subcore_barrier()

        ebase = cid * E_PER_CORE + sid * E_PER_SUB
        lanes = lax.iota(jnp.int32, 16)
        hmask = lanes < heads

        def chunk_body(i, carry):
            base = ebase + i * CHUNK
            pltpu.sync_copy(src_hbm.at[pl.ds(base, CHUNK)], src_v)
            pltpu.sync_copy(dst_hbm.at[pl.ds(base, CHUNK)], dst_v)
            ch = pltpu.async_copy(h_hbm.at[src_v], hsrc_v, sem_h)
            ca = pltpu.async_copy(sda_hbm.at[src_v], sa_v, sem_a)
            cb = pltpu.async_copy(sdb_hbm.at[dst_v], sb_v, sem_b)
            ch.wait()
            ca.wait()
            cb.wait()

            def edge_body(i, ecarry):
                for u in range(UNROLL):
                    j = i * UNROLL + u
                    e = sa_v[j, :] + sb_v[j, :]
                    e = jnp.where(e > 0, e, NEG * e)
                    w = jnp.exp(e)
                    msg_v[j, pl.ds(F, 16)] = jnp.where(hmask, w, 0.0)
                    for g in range(8):
                        wspl = _splat(w, grp_head[g])
                        msg_v[j, pl.ds(16 * g, 16)] = (
                            hsrc_v[j, pl.ds(16 * g, 16)] * wspl)
                return ecarry

            lax.fori_loop(0, CHUNK // UNROLL, edge_body, 0)
            pltpu.sync_copy(msg_v, acc.at[dst_v], add=True)
            return carry

        lax.fori_loop(0, NCHUNK, chunk_body, 0)

        plsc.subcore_barrier()
        pltpu.sync_copy(acc.at[pl.ds(row0, ROWS_PER_SUB)],
                        out_hbm.at[cid, pl.ds(row0, ROWS_PER_SUB)])

    return sc_agg


_SC_CACHE = {}


def _sc_agg(heads):
    if heads not in _SC_CACHE:
        _SC_CACHE[heads] = _make_sc_agg(heads)
    return _SC_CACHE[heads]


_GRID = 10
_BLK = N // _GRID


def _row_spec(w):
    return pl.BlockSpec((_BLK, w), lambda i: (i, 0))


def _full_spec(shape):
    return pl.BlockSpec(shape, lambda i: tuple(0 for _ in shape))


def _tc1_body(x_ref, w1_ref, ma_ref, mb_ref, h_ref, sda_ref, sdb_ref):
    h = _dot(x_ref[...], w1_ref[...])
    h_ref[...] = h
    sda_ref[...] = _dot(h, ma_ref[...])
    sdb_ref[...] = _dot(h, mb_ref[...])


def _tc1(x, w1, ma, mb):
    return pl.pallas_call(
        _tc1_body,
        grid=(_GRID,),
        in_specs=[_row_spec(F), _full_spec((F, F)),
                  _full_spec((F, 16)), _full_spec((F, 16))],
        out_specs=[_row_spec(F), _row_spec(16), _row_spec(16)],
        out_shape=[
            jax.ShapeDtypeStruct((N, F), jnp.float32),
            jax.ShapeDtypeStruct((N, 16), jnp.float32),
            jax.ShapeDtypeStruct((N, 16), jnp.float32),
        ],
    )(x, w1, ma, mb)


def _tc2_body(p_ref, h1_ref, sda1_ref, b1_ref, r_ref, w2_ref, ma2_ref,
              mb2_ref, h2_ref, sda2_ref, sdb2_ref):
    p = p_ref[0] + p_ref[1]
    num = p[:, 0:F]
    den8 = p[:, F:F + 8]
    es = sda1_ref[:, 0:8] + sda1_ref[:, 8:16]
    es = jnp.where(es > 0, es, NEG * es)
    wself = jnp.exp(es)
    r = r_ref[...]
    den_exp = _dot(den8 + wself, r)
    wself_exp = _dot(wself, r)
    out = (num + h1_ref[...] * wself_exp) / (den_exp + 1e-16) + b1_ref[...]
    act = jnp.where(out > 0, out, jnp.exp(out) - 1.0)
    h2 = _dot(act, w2_ref[...])
    h2_ref[...] = h2
    sda2_ref[...] = _dot(h2, ma2_ref[...])
    sdb2_ref[...] = _dot(h2, mb2_ref[...])


def _tc2(p, h1, sda1, b1, r, w2, ma2, mb2):
    return pl.pallas_call(
        _tc2_body,
        grid=(_GRID,),
        in_specs=[
            pl.BlockSpec((NUM_CORES, _BLK, ACC_W), lambda i: (0, i, 0)),
            _row_spec(F), _row_spec(16), _full_spec((1, F)),
            _full_spec((8, F)), _full_spec((F, F)),
            _full_spec((F, 16)), _full_spec((F, 16)),
        ],
        out_specs=[_row_spec(F), _row_spec(16), _row_spec(16)],
        out_shape=[
            jax.ShapeDtypeStruct((N, F), jnp.float32),
            jax.ShapeDtypeStruct((N, 16), jnp.float32),
            jax.ShapeDtypeStruct((N, 16), jnp.float32),
        ],
    )(p, h1, sda1, b1, r, w2, ma2, mb2)


def _tc3_body(p_ref, h2_ref, sda2_ref, b2_ref, bng_ref, bnb_ref, lng_ref,
              lnb_ref, y_ref):
    p = p_ref[0] + p_ref[1]
    num = p[:, 0:F]
    den = p[:, F:F + 1]
    es = sda2_ref[:, 0:1] + sda2_ref[:, 8:9]
    es = jnp.where(es > 0, es, NEG * es)
    wself = jnp.exp(es)
    out = (num + h2_ref[...] * wself) / (den + wself + 1e-16) + b2_ref[...]
    act = jnp.where(out > 0, out, jnp.exp(out) - 1.0)
    hb = act * bng_ref[...] + bnb_ref[...]
    mu = jnp.mean(hb, axis=-1, keepdims=True)
    var = jnp.mean((hb - mu) * (hb - mu), axis=-1, keepdims=True)
    y_ref[...] = (hb - mu) / jnp.sqrt(var + EPS) * lng_ref[...] + lnb_ref[...]


def _tc3(p, h2, sda2, b2, bng, bnb, lng, lnb):
    return pl.pallas_call(
        _tc3_body,
        grid=(_GRID,),
        in_specs=[
            pl.BlockSpec((NUM_CORES, _BLK, ACC_W), lambda i: (0, i, 0)),
            _row_spec(F), _row_spec(16),
            _full_spec((1, F)), _full_spec((1, F)), _full_spec((1, F)),
            _full_spec((1, F)), _full_spec((1, F)),
        ],
        out_specs=_row_spec(F),
        out_shape=jax.ShapeDtypeStruct((N, F), jnp.float32),
    )(p, h2, sda2, b2, bng, bnb, lng, lnb)


def _logit_mats(a_src, a_dst, heads):
    chan = F // heads
    eye = jnp.eye(heads, dtype=jnp.float32)
    a_s = (a_src[:, :, None] * eye[:, None, :]).reshape(F, heads)
    a_d = (a_dst[:, :, None] * eye[:, None, :]).reshape(F, heads)
    pad = jnp.zeros((F, 8 - heads), jnp.float32)
    a_s = jnp.concatenate([a_s, pad], axis=1)
    a_d = jnp.concatenate([a_d, pad], axis=1)
    ma = jnp.concatenate([a_s, a_d], axis=1)
    mb = jnp.concatenate([a_d, a_s], axis=1)
    return ma, mb


def kernel(x, edge_index, W1, a_src1, a_dst1, b1, W2, a_src2, a_dst2, b2,
           bn_gamma, bn_beta, ln_gamma, ln_beta):
    src = edge_index[0].astype(jnp.int32)
    dst = edge_index[1].astype(jnp.int32)
    ma1, mb1 = _logit_mats(a_src1, a_dst1, 8)
    ma2, mb2 = _logit_mats(a_src2, a_dst2, 1)
    r = (jnp.eye(8, dtype=jnp.float32)[:, :, None]
         * jnp.ones((1, 1, 16), jnp.float32)).reshape(8, F)
    zeros_acc = jnp.zeros((N_PAD, ACC_W), jnp.float32)

    h1, sda1, sdb1 = _tc1(x, W1, ma1, mb1)
    p1 = _sc_agg(8)(h1, sda1, sdb1, src, dst, zeros_acc)
    h2, sda2, sdb2 = _tc2(p1, h1, sda1, b1.reshape(1, F), r, W2, ma2, mb2)
    p2 = _sc_agg(1)(h2, sda2, sdb2, src, dst, zeros_acc)
    return _tc3(p2, h2, sda2, b2.reshape(1, F), bn_gamma.reshape(1, F),
                bn_beta.reshape(1, F), ln_gamma.reshape(1, F),
                ln_beta.reshape(1, F))


# speedup vs baseline: 42.2255x; 1.4687x over previous
import functools

import jax
import jax.numpy as jnp
from jax import lax
from jax.experimental import pallas as pl
from jax.experimental.pallas import tpu as pltpu
from jax.experimental.pallas import tpu_sc as plsc

N = 10000
E = 320000
F = 128
ACC_W = 144
NEG = 0.2
EPS = 1e-5

NUM_CORES = 2
NUM_SUBCORES = 16
E_PER_CORE = E // NUM_CORES
E_PER_SUB = E_PER_CORE // NUM_SUBCORES
CHUNK = 40
UNROLL = 8
NCHUNK = E_PER_SUB // CHUNK
N_PAD = 10000
ROWS_A = 632
ROWS_B = N_PAD - 15 * ROWS_A

_HIGH = lax.Precision.HIGHEST

_GATHER_DNUMS = lax.GatherDimensionNumbers(
    offset_dims=(), collapsed_slice_dims=(0,), start_index_map=(0,))


def _splat(v, lane):
    idx = jnp.full((16, 1), lane, jnp.int32)
    return lax.gather(v, idx, _GATHER_DNUMS, slice_sizes=(1,),
                      mode=lax.GatherScatterMode.PROMISE_IN_BOUNDS)


def _dot(a, b):
    return jnp.dot(a, b, preferred_element_type=jnp.float32, precision=_HIGH)


def _make_sc_agg(heads):
    chan = F // heads
    grp_head = [(16 * g) // chan for g in range(8)]
    mesh = plsc.VectorSubcoreMesh(core_axis_name="c", subcore_axis_name="s")

    @functools.partial(
        pl.kernel,
        out_type=jax.ShapeDtypeStruct((NUM_CORES, N_PAD, ACC_W), jnp.float32),
        mesh=mesh,
        compiler_params=pltpu.CompilerParams(use_tc_tiling_on_sc=False),
        scratch_types=[
            pltpu.VMEM_SHARED((N_PAD, ACC_W), jnp.float32),
            pltpu.VMEM((CHUNK,), jnp.int32),
            pltpu.VMEM((CHUNK,), jnp.int32),
            pltpu.VMEM((CHUNK,), jnp.int32),
            pltpu.VMEM((CHUNK,), jnp.int32),
            pltpu.VMEM((CHUNK,), jnp.int32),
            pltpu.VMEM((CHUNK,), jnp.int32),
            pltpu.VMEM((CHUNK,), jnp.int32),
            pltpu.VMEM((CHUNK,), jnp.int32),
            pltpu.VMEM((CHUNK, F), jnp.float32),
            pltpu.VMEM((CHUNK, F), jnp.float32),
            pltpu.VMEM((CHUNK, 16), jnp.float32),
            pltpu.VMEM((CHUNK, 16), jnp.float32),
            pltpu.VMEM((CHUNK, 16), jnp.float32),
            pltpu.VMEM((CHUNK, 16), jnp.float32),
            pltpu.VMEM((CHUNK, ACC_W), jnp.float32),
            pltpu.VMEM((CHUNK, ACC_W), jnp.float32),
            pltpu.SemaphoreType.DMA,
            pltpu.SemaphoreType.DMA,
            pltpu.SemaphoreType.DMA,
            pltpu.SemaphoreType.DMA,
            pltpu.SemaphoreType.DMA,
            pltpu.SemaphoreType.DMA,
            pltpu.SemaphoreType.DMA,
            pltpu.SemaphoreType.DMA,
        ],
    )
    def sc_agg(h_hbm, sda_hbm, sdb_hbm, src_hbm, dst_hbm, zero_hbm,
               out_hbm, acc, s4_0, s4_1, s4_2, s4_3, d4_0, d4_1, d4_2, d4_3,
               hs0, hs1, sa0, sa1, sb0, sb1, mg0, mg1,
               si0, si1, si2, si3, sg0, sg1, ss0, ss1):
        cid = lax.axis_index("c")
        sid = lax.axis_index("s")
        row0 = sid * ROWS_A
        ebase = cid * E_PER_CORE + sid * E_PER_SUB

        def slice_rows(ref_fn):
            @pl.when(sid < 15)
            def _():
                ref_fn(row0, ROWS_A)

            @pl.when(sid == 15)
            def _():
                ref_fn(15 * ROWS_A, ROWS_B)
        lanes = lax.iota(jnp.int32, 16)
        hmask = lanes < heads
        s4 = (s4_0, s4_1, s4_2, s4_3)
        d4 = (d4_0, d4_1, d4_2, d4_3)
        si = (si0, si1, si2, si3)
        hs = (hs0, hs1)
        sa = (sa0, sa1)
        sb = (sb0, sb1)
        mg = (mg0, mg1)
        sg = (sg0, sg1)
        ss = (ss0, ss1)

        def start_idx(m4, ci):
            base = ebase + ci * CHUNK
            pltpu.async_copy(src_hbm.at[pl.ds(base, CHUNK)], s4[m4], si[m4])
            pltpu.async_copy(dst_hbm.at[pl.ds(base, CHUNK)], d4[m4], si[m4])

        def wait_idx(m4):
            pltpu.make_async_copy(
                src_hbm.at[pl.ds(0, CHUNK)], s4[m4], si[m4]).wait()
            pltpu.make_async_copy(
                dst_hbm.at[pl.ds(0, CHUNK)], d4[m4], si[m4]).wait()

        def start_gathers(b, m4):
            pltpu.async_copy(h_hbm.at[s4[m4]], hs[b], sg[b])
            pltpu.async_copy(sda_hbm.at[s4[m4]], sa[b], sg[b])
            pltpu.async_copy(sdb_hbm.at[d4[m4]], sb[b], sg[b])

        def wait_gathers(b, m4):
            pltpu.make_async_copy(h_hbm.at[s4[m4]], hs[b], sg[b]).wait()
            pltpu.make_async_copy(sda_hbm.at[s4[m4]], sa[b], sg[b]).wait()
            pltpu.make_async_copy(sdb_hbm.at[d4[m4]], sb[b], sg[b]).wait()

        def start_scatter(b, m4):
            pltpu.async_copy(mg[b], acc.at[d4[m4]], ss[b], add=True)

        def wait_scatter(b, m4):
            pltpu.make_async_copy(mg[b], acc.at[d4[m4]], ss[b]).wait()

        def compute(b):
            hs_v, sa_v, sb_v, msg_v = hs[b], sa[b], sb[b], mg[b]

            def edge_body(i, ecarry):
                for u in range(UNROLL):
                    j = i * UNROLL + u
                    e = sa_v[j, :] + sb_v[j, :]
                    e = jnp.where(e > 0, e, NEG * e)
                    w = jnp.exp(e)
                    msg_v[j, pl.ds(F, 16)] = jnp.where(hmask, w, 0.0)
                    for g in range(8):
                        wspl = _splat(w, grp_head[g])
                        msg_v[j, pl.ds(16 * g, 16)] = (
                            hs_v[j, pl.ds(16 * g, 16)] * wspl)
                return ecarry

            lax.fori_loop(0, CHUNK // UNROLL, edge_body, 0)

        start_idx(0, 0)
        start_idx(1, 1)
        wait_idx(0)
        start_gathers(0, 0)
        slice_rows(lambda r0, nr: pltpu.sync_copy(
            zero_hbm.at[pl.ds(r0, nr)], acc.at[pl.ds(r0, nr)]))
        plsc.subcore_barrier()

        def quad_body(q, carry):
            for u in range(4):
                b = u % 2
                ci = 4 * q + u

                @pl.when(ci >= 2)
                def _():
                    wait_scatter(b, (u + 2) % 4)

                @pl.when(ci + 2 < NCHUNK)
                def _():
                    start_idx((u + 2) % 4, ci + 2)

                @pl.when(ci + 1 < NCHUNK)
                def _():
                    wait_idx((u + 1) % 4)
                    start_gathers(1 - b, (u + 1) % 4)

                @pl.when(ci < NCHUNK)
                def _():
                    wait_gathers(b, u)
                    compute(b)
                    start_scatter(b, u)
            return carry

        lax.fori_loop(0, (NCHUNK + 3) // 4, quad_body, 0)

        plsc.subcore_barrier()
        slice_rows(lambda r0, nr: pltpu.sync_copy(
            acc.at[pl.ds(r0, nr)], out_hbm.at[cid, pl.ds(r0, nr)]))

    return sc_agg


_SC_CACHE = {}


def _sc_agg(heads):
    if heads not in _SC_CACHE:
        _SC_CACHE[heads] = _make_sc_agg(heads)
    return _SC_CACHE[heads]


_GRID = 10
_BLK = N // _GRID


def _row_spec(w):
    return pl.BlockSpec((_BLK, w), lambda i: (i, 0))


def _full_spec(shape):
    return pl.BlockSpec(shape, lambda i: tuple(0 for _ in shape))


def _tc1_body(x_ref, w1_ref, ma_ref, mb_ref, h_ref, sda_ref, sdb_ref):
    h = _dot(x_ref[...], w1_ref[...])
    h_ref[...] = h
    sda_ref[...] = _dot(h, ma_ref[...])
    sdb_ref[...] = _dot(h, mb_ref[...])


def _tc1(x, w1, ma, mb):
    return pl.pallas_call(
        _tc1_body,
        grid=(_GRID,),
        in_specs=[_row_spec(F), _full_spec((F, F)),
                  _full_spec((F, 16)), _full_spec((F, 16))],
        out_specs=[_row_spec(F), _row_spec(16), _row_spec(16)],
        out_shape=[
            jax.ShapeDtypeStruct((N, F), jnp.float32),
            jax.ShapeDtypeStruct((N, 16), jnp.float32),
            jax.ShapeDtypeStruct((N, 16), jnp.float32),
        ],
    )(x, w1, ma, mb)


def _tc2_body(p_ref, h1_ref, sda1_ref, b1_ref, r_ref, w2_ref, ma2_ref,
              mb2_ref, h2_ref, sda2_ref, sdb2_ref):
    p = p_ref[0] + p_ref[1]
    num = p[:, 0:F]
    den8 = p[:, F:F + 8]
    es = sda1_ref[:, 0:8] + sda1_ref[:, 8:16]
    es = jnp.where(es > 0, es, NEG * es)
    wself = jnp.exp(es)
    r = r_ref[...]
    den_exp = _dot(den8 + wself, r)
    wself_exp = _dot(wself, r)
    out = (num + h1_ref[...] * wself_exp) / (den_exp + 1e-16) + b1_ref[...]
    act = jnp.where(out > 0, out, jnp.exp(out) - 1.0)
    h2 = _dot(act, w2_ref[...])
    h2_ref[...] = h2
    sda2_ref[...] = _dot(h2, ma2_ref[...])
    sdb2_ref[...] = _dot(h2, mb2_ref[...])


def _tc2(p, h1, sda1, b1, r, w2, ma2, mb2):
    return pl.pallas_call(
        _tc2_body,
        grid=(_GRID,),
        in_specs=[
            pl.BlockSpec((NUM_CORES, _BLK, ACC_W), lambda i: (0, i, 0)),
            _row_spec(F), _row_spec(16), _full_spec((1, F)),
            _full_spec((8, F)), _full_spec((F, F)),
            _full_spec((F, 16)), _full_spec((F, 16)),
        ],
        out_specs=[_row_spec(F), _row_spec(16), _row_spec(16)],
        out_shape=[
            jax.ShapeDtypeStruct((N, F), jnp.float32),
            jax.ShapeDtypeStruct((N, 16), jnp.float32),
            jax.ShapeDtypeStruct((N, 16), jnp.float32),
        ],
    )(p, h1, sda1, b1, r, w2, ma2, mb2)


def _tc3_body(p_ref, h2_ref, sda2_ref, b2_ref, bng_ref, bnb_ref, lng_ref,
              lnb_ref, y_ref):
    p = p_ref[0] + p_ref[1]
    num = p[:, 0:F]
    den = p[:, F:F + 1]
    es = sda2_ref[:, 0:1] + sda2_ref[:, 8:9]
    es = jnp.where(es > 0, es, NEG * es)
    wself = jnp.exp(es)
    out = (num + h2_ref[...] * wself) / (den + wself + 1e-16) + b2_ref[...]
    act = jnp.where(out > 0, out, jnp.exp(out) - 1.0)
    hb = act * bng_ref[...] + bnb_ref[...]
    mu = jnp.mean(hb, axis=-1, keepdims=True)
    var = jnp.mean((hb - mu) * (hb - mu), axis=-1, keepdims=True)
    y_ref[...] = (hb - mu) / jnp.sqrt(var + EPS) * lng_ref[...] + lnb_ref[...]


def _tc3(p, h2, sda2, b2, bng, bnb, lng, lnb):
    return pl.pallas_call(
        _tc3_body,
        grid=(_GRID,),
        in_specs=[
            pl.BlockSpec((NUM_CORES, _BLK, ACC_W), lambda i: (0, i, 0)),
            _row_spec(F), _row_spec(16),
            _full_spec((1, F)), _full_spec((1, F)), _full_spec((1, F)),
            _full_spec((1, F)), _full_spec((1, F)),
        ],
        out_specs=_row_spec(F),
        out_shape=jax.ShapeDtypeStruct((N, F), jnp.float32),
    )(p, h2, sda2, b2, bng, bnb, lng, lnb)


def _logit_mats(a_src, a_dst, heads):
    chan = F // heads
    eye = jnp.eye(heads, dtype=jnp.float32)
    a_s = (a_src[:, :, None] * eye[:, None, :]).reshape(F, heads)
    a_d = (a_dst[:, :, None] * eye[:, None, :]).reshape(F, heads)
    pad = jnp.zeros((F, 8 - heads), jnp.float32)
    a_s = jnp.concatenate([a_s, pad], axis=1)
    a_d = jnp.concatenate([a_d, pad], axis=1)
    ma = jnp.concatenate([a_s, a_d], axis=1)
    mb = jnp.concatenate([a_d, a_s], axis=1)
    return ma, mb


def kernel(x, edge_index, W1, a_src1, a_dst1, b1, W2, a_src2, a_dst2, b2,
           bn_gamma, bn_beta, ln_gamma, ln_beta):
    src = edge_index[0].astype(jnp.int32)
    dst = edge_index[1].astype(jnp.int32)
    ma1, mb1 = _logit_mats(a_src1, a_dst1, 8)
    ma2, mb2 = _logit_mats(a_src2, a_dst2, 1)
    r = (jnp.eye(8, dtype=jnp.float32)[:, :, None]
         * jnp.ones((1, 1, 16), jnp.float32)).reshape(8, F)
    zeros_acc = jnp.zeros((N_PAD, ACC_W), jnp.float32)

    h1, sda1, sdb1 = _tc1(x, W1, ma1, mb1)
    p1 = _sc_agg(8)(h1, sda1, sdb1, src, dst, zeros_acc)
    h2, sda2, sdb2 = _tc2(p1, h1, sda1, b1.reshape(1, F), r, W2, ma2, mb2)
    p2 = _sc_agg(1)(h2, sda2, sdb2, src, dst, zeros_acc)
    return _tc3(p2, h2, sda2, b2.reshape(1, F), bn_gamma.reshape(1, F),
                bn_beta.reshape(1, F), ln_gamma.reshape(1, F),
                ln_beta.reshape(1, F))
